# Initial kernel scaffold; baseline (speedup 1.0000x reference)
import jax
import jax.numpy as jnp
from jax.experimental import pallas as pl


def kernel(logits):
    raise NotImplementedError("write your pallas kernel here")


# speedup vs baseline: 2.5331x; 2.5331x over previous
import jax
import jax.numpy as jnp
from jax.experimental import pallas as pl

_TOP_K = 3
_EPS = 2.220446049250313e-16
_NEG_INF = float("-inf")


def _beam_kernel(x_ref, tok_ref, sc_ref):
    L, B, V = x_ref.shape
    K = _TOP_K
    iota_v = jax.lax.broadcasted_iota(jnp.int32, (B, V), 1)
    iota_c = jax.lax.broadcasted_iota(jnp.int32, (B, L), 1)

    def logp_row(t):
        x = x_ref[t]
        m = jnp.max(x, axis=1, keepdims=True)
        e = jnp.exp(x - m)
        s = jnp.sum(e, axis=1, keepdims=True)
        return jnp.log(e / s + _EPS)

    work = logp_row(0)
    scores, seqs = [], []
    for j in range(K):
        v = jnp.max(work, axis=1, keepdims=True)
        idx = jnp.min(jnp.where(work == v, iota_v, V), axis=1, keepdims=True)
        scores.append(v)
        seqs.append(jnp.where(iota_c == 0, idx, 0))
        work = jnp.where(iota_v == idx, _NEG_INF, work)

    for t in range(1, L):
        lp = logp_row(t)
        cand = [scores[k] + lp for k in range(K)]
        vals, beams, tks = [], [], []
        for j in range(K):
            vmax = [jnp.max(cand[k], axis=1, keepdims=True) for k in range(K)]
            tkk = [jnp.min(jnp.where(cand[k] == vmax[k], iota_v, V),
                           axis=1, keepdims=True) for k in range(K)]
            best = jnp.maximum(jnp.maximum(vmax[0], vmax[1]), vmax[2])
            beam = jnp.where(vmax[0] == best, 0,
                             jnp.where(vmax[1] == best, 1, 2))
            tok = jnp.where(vmax[0] == best, tkk[0],
                            jnp.where(vmax[1] == best, tkk[1], tkk[2]))
            vals.append(best)
            beams.append(beam)
            tks.append(tok)
            for k in range(K):
                cand[k] = jnp.where((beam == k) & (iota_v == tok),
                                    _NEG_INF, cand[k])
        new_scores, new_seqs = [], []
        for j in range(K):
            g = jnp.zeros((B, L), jnp.int32)
            for k in range(K):
                g = g + jnp.where(beams[j] == k, 1, 0) * seqs[k]
            g = jnp.where(iota_c == t, tks[j], g)
            new_scores.append(vals[j])
            new_seqs.append(g)
        scores, seqs = new_scores, new_seqs

    for j in range(K):
        tok_ref[j] = seqs[j]
    sc_ref[...] = jnp.concatenate(scores, axis=1)


def kernel(logits):
    B, L, V = logits.shape
    x = logits.transpose(1, 0, 2)
    toks3, scores = pl.pallas_call(
        _beam_kernel,
        out_shape=(
            jax.ShapeDtypeStruct((_TOP_K, B, L), jnp.int32),
            jax.ShapeDtypeStruct((B, _TOP_K), jnp.float32),
        ),
    )(x)
    return toks3.transpose(1, 2, 0), scores


# speedup vs baseline: 3.1310x; 1.2360x over previous
import jax
import jax.numpy as jnp
from jax.experimental import pallas as pl

_TOP_K = 3
_EPS = 2.220446049250313e-16
_NEG_INF = float("-inf")


def _beam_kernel(x_ref, tok_ref, sc_ref):
    B, L, V = x_ref.shape
    K = _TOP_K

    x32 = jnp.concatenate([x_ref[:, t, :] for t in range(L)], axis=0)

    m = jnp.max(x32, axis=1, keepdims=True)
    e = jnp.exp(x32 - m)
    s = jnp.sum(e, axis=1, keepdims=True)
    lp = jnp.log(e / s + _EPS)

    iota_v = jax.lax.broadcasted_iota(jnp.int32, (L * B, V), 1)
    vals, toks = [], []
    work = lp
    for i in range(K):
        v = jnp.max(work, axis=1, keepdims=True)
        idx = jnp.min(jnp.where(work == v, iota_v, V), axis=1, keepdims=True)
        vals.append(v)
        toks.append(idx)
        if i + 1 < K:
            work = jnp.where(iota_v == idx, _NEG_INF, work)

    def step_slice(a, t):
        return a[t * B:(t + 1) * B]

    scores = [step_slice(vals[i], 0) for i in range(K)]
    iota_c = jax.lax.broadcasted_iota(jnp.int32, (B, L), 1)
    seqs = [jnp.where(iota_c == 0, step_slice(toks[i], 0), 0)
            for i in range(K)]

    iota9 = jax.lax.broadcasted_iota(jnp.int32, (B, K * K), 1)
    for t in range(1, L):
        v_t = [step_slice(vals[i], t) for i in range(K)]
        tok_t = [step_slice(toks[i], t) for i in range(K)]
        cand9 = jnp.concatenate(
            [scores[k] + v_t[i] for k in range(K) for i in range(K)], axis=1)
        new_scores, new_seqs = [], []
        for j in range(K):
            best = jnp.max(cand9, axis=1, keepdims=True)
            p = jnp.min(jnp.where(cand9 == best, iota9, K * K),
                        axis=1, keepdims=True)
            beam = p // K
            irank = p - K * beam
            tok = jnp.zeros((B, 1), jnp.int32)
            g = jnp.zeros((B, L), jnp.int32)
            for k in range(K):
                tok = tok + jnp.where(irank == k, tok_t[k], 0)
                g = g + jnp.where(beam == k, 1, 0) * seqs[k]
            g = jnp.where(iota_c == t, tok, g)
            new_scores.append(best)
            new_seqs.append(g)
            if j + 1 < K:
                cand9 = jnp.where(iota9 == p, _NEG_INF, cand9)
        scores, seqs = new_scores, new_seqs

    tok_ref[...] = jnp.stack(seqs, axis=-1).astype(jnp.int32)
    sc_ref[...] = jnp.concatenate(scores, axis=1)


def kernel(logits):
    B, L, V = logits.shape
    return pl.pallas_call(
        _beam_kernel,
        out_shape=(
            jax.ShapeDtypeStruct((B, L, _TOP_K), jnp.int32),
            jax.ShapeDtypeStruct((B, _TOP_K), jnp.float32),
        ),
    )(logits)
